# Initial kernel scaffold; baseline (speedup 1.0000x reference)
import jax
import jax.numpy as jnp
from jax.experimental import pallas as pl


def kernel(pos, feat, n_point, W1, b1, gamma1, beta1, W2, b2, gamma2, beta2):
    raise NotImplementedError("write your pallas kernel here")


# speedup vs baseline: 1.0013x; 1.0013x over previous
import functools
from typing import Any

import jax
import jax.numpy as jnp
import numpy as np
from jax.experimental import pallas as pl
from jax.experimental.pallas import tpu as pltpu

N_NEIGHBOR = 64
N_POINT_STATIC = 2048


def _fps_jax(pos, n_point_static):
    pos = jax.lax.stop_gradient(pos)
    B, N, _ = pos.shape

    def body(i, state):
        centroids, dists, farthest = state
        centroids = centroids.at[:, i].set(farthest)
        cpos = jnp.take_along_axis(pos, farthest[:, None, None].astype(jnp.int32), axis=1)
        d = jnp.sum((pos - cpos) ** 2, axis=-1)
        dists = jnp.minimum(dists, d)
        farthest = jnp.argmax(dists, axis=-1).astype(jnp.int32)
        return (centroids, dists, farthest)

    centroids = jnp.zeros((B, n_point_static), dtype=jnp.int32)
    dists = jnp.full((B, N), 1e10, dtype=jnp.float32)
    farthest = jnp.zeros((B,), dtype=jnp.int32)
    centroids, _, _ = jax.lax.fori_loop(0, n_point_static, body, (centroids, dists, farthest))
    return centroids


_MB = 128


def _stats1_body(nbr_ref, cen_ref, wn_ref, w1a_ref, b1_ref, sum_ref, ssq_ref,
                 acc_s, acc_q):
    b = pl.program_id(0)
    m = pl.program_id(1)
    step = b * pl.num_programs(1) + m
    nbr = nbr_ref[0]
    cen = cen_ref[0]
    wn = wn_ref[...]
    w1a = w1a_ref[...]
    rows = nbr.reshape(_MB * N_NEIGHBOR, nbr.shape[-1])
    h = jnp.dot(rows, wn, preferred_element_type=jnp.float32)
    bias = b1_ref[...] - jnp.dot(cen, w1a, preferred_element_type=jnp.float32)
    h = h.reshape(_MB, N_NEIGHBOR, h.shape[-1]) + bias[:, None, :]
    s = jnp.sum(h, axis=(0, 1), keepdims=False)[None, :]
    q = jnp.sum(h * h, axis=(0, 1), keepdims=False)[None, :]

    @pl.when(step == 0)
    def _():
        acc_s[...] = jnp.zeros_like(acc_s)
        acc_q[...] = jnp.zeros_like(acc_q)

    acc_s[0:1, :] += s
    acc_q[0:1, :] += q

    @pl.when(step == pl.num_programs(0) * pl.num_programs(1) - 1)
    def _():
        sum_ref[...] = acc_s[0:1, :]
        ssq_ref[...] = acc_q[0:1, :]


def _layer2_body(nbr_ref, cen_ref, wn_ref, w1a_ref, b1_ref, mu1_ref, is1_ref,
                 w2_ref, b2_ref, hmax_ref, hmin_ref, sum_ref, ssq_ref,
                 acc_s, acc_q):
    b = pl.program_id(0)
    m = pl.program_id(1)
    step = b * pl.num_programs(1) + m
    nbr = nbr_ref[0]
    cen = cen_ref[0]
    rows = nbr.reshape(_MB * N_NEIGHBOR, nbr.shape[-1])
    h = jnp.dot(rows, wn_ref[...], preferred_element_type=jnp.float32)
    bias = b1_ref[...] - jnp.dot(cen, w1a_ref[...], preferred_element_type=jnp.float32)
    h = h.reshape(_MB, N_NEIGHBOR, h.shape[-1]) + bias[:, None, :]
    h = jnp.maximum(h * is1_ref[...][None, :] + mu1_ref[...][None, :], 0.0)
    h2 = jnp.dot(h.reshape(_MB * N_NEIGHBOR, h.shape[-1]), w2_ref[...],
                 preferred_element_type=jnp.float32) + b2_ref[...]
    s = jnp.sum(h2, axis=0)[None, :]
    q = jnp.sum(h2 * h2, axis=0)[None, :]
    h2 = h2.reshape(_MB, N_NEIGHBOR, h2.shape[-1])
    hmax_ref[0] = jnp.max(h2, axis=1)
    hmin_ref[0] = jnp.min(h2, axis=1)

    @pl.when(step == 0)
    def _():
        acc_s[...] = jnp.zeros_like(acc_s)
        acc_q[...] = jnp.zeros_like(acc_q)

    acc_s[0:1, :] += s
    acc_q[0:1, :] += q

    @pl.when(step == pl.num_programs(0) * pl.num_programs(1) - 1)
    def _():
        sum_ref[...] = acc_s[0:1, :]
        ssq_ref[...] = acc_q[0:1, :]


def _final_body(hmax_ref, hmin_ref, mu2_ref, is2_ref, out_ref):
    a = hmax_ref[...] * is2_ref[...] + mu2_ref[...]
    c = hmin_ref[...] * is2_ref[...] + mu2_ref[...]
    out_ref[...] = jnp.maximum(jnp.maximum(a, c), 0.0)


def _knnconv(nbr_feat, center_feat, W1, b1, gamma1, beta1, W2, b2, gamma2, beta2):
    B, M, K, C = nbr_feat.shape
    C_out = W1.shape[0]
    W1a = W1[:, :C]
    W1b = W1[:, C:]
    Wn = (W1a + W1b).T
    W1aT = W1a.T
    count = float(B * M * K)

    grid = (B, M // _MB)
    stats = pl.pallas_call(
        _stats1_body,
        grid=grid,
        in_specs=[
            pl.BlockSpec((1, _MB, K, C), lambda b, m: (b, m, 0, 0)),
            pl.BlockSpec((1, _MB, C), lambda b, m: (b, m, 0)),
            pl.BlockSpec((C, C_out), lambda b, m: (0, 0)),
            pl.BlockSpec((C, C_out), lambda b, m: (0, 0)),
            pl.BlockSpec((C_out,), lambda b, m: (0,)),
        ],
        out_specs=[
            pl.BlockSpec((1, C_out), lambda b, m: (0, 0)),
            pl.BlockSpec((1, C_out), lambda b, m: (0, 0)),
        ],
        out_shape=[
            jax.ShapeDtypeStruct((1, C_out), jnp.float32),
            jax.ShapeDtypeStruct((1, C_out), jnp.float32),
        ],
        scratch_shapes=[
            pltpu.VMEM((8, C_out), jnp.float32),
            pltpu.VMEM((8, C_out), jnp.float32),
        ],
    )(nbr_feat, center_feat, Wn, W1aT, b1)
    s1, q1 = stats[0][0], stats[1][0]
    mean1 = s1 / count
    var1 = q1 / count - mean1 * mean1
    inv1 = gamma1 / jnp.sqrt(var1 + 1e-5)
    mu1 = beta1 - mean1 * inv1

    hmax, hmin, s2m, q2m = pl.pallas_call(
        _layer2_body,
        grid=grid,
        in_specs=[
            pl.BlockSpec((1, _MB, K, C), lambda b, m: (b, m, 0, 0)),
            pl.BlockSpec((1, _MB, C), lambda b, m: (b, m, 0)),
            pl.BlockSpec((C, C_out), lambda b, m: (0, 0)),
            pl.BlockSpec((C, C_out), lambda b, m: (0, 0)),
            pl.BlockSpec((C_out,), lambda b, m: (0,)),
            pl.BlockSpec((C_out,), lambda b, m: (0,)),
            pl.BlockSpec((C_out,), lambda b, m: (0,)),
            pl.BlockSpec((C_out, C_out), lambda b, m: (0, 0)),
            pl.BlockSpec((C_out,), lambda b, m: (0,)),
        ],
        out_specs=[
            pl.BlockSpec((1, _MB, C_out), lambda b, m: (b, m, 0)),
            pl.BlockSpec((1, _MB, C_out), lambda b, m: (b, m, 0)),
            pl.BlockSpec((1, C_out), lambda b, m: (0, 0)),
            pl.BlockSpec((1, C_out), lambda b, m: (0, 0)),
        ],
        out_shape=[
            jax.ShapeDtypeStruct((B, M, C_out), jnp.float32),
            jax.ShapeDtypeStruct((B, M, C_out), jnp.float32),
            jax.ShapeDtypeStruct((1, C_out), jnp.float32),
            jax.ShapeDtypeStruct((1, C_out), jnp.float32),
        ],
        scratch_shapes=[
            pltpu.VMEM((8, C_out), jnp.float32),
            pltpu.VMEM((8, C_out), jnp.float32),
        ],
    )(nbr_feat, center_feat, Wn, W1aT, b1, mu1, inv1, W2.T, b2)
    s2, q2 = s2m[0], q2m[0]
    mean2 = s2 / count
    var2 = q2 / count - mean2 * mean2
    inv2 = gamma2 / jnp.sqrt(var2 + 1e-5)
    mu2 = beta2 - mean2 * inv2

    out = pl.pallas_call(
        _final_body,
        grid=(B,),
        in_specs=[
            pl.BlockSpec((1, M, C_out), lambda b: (b, 0, 0)),
            pl.BlockSpec((1, M, C_out), lambda b: (b, 0, 0)),
            pl.BlockSpec((C_out,), lambda b: (0,)),
            pl.BlockSpec((C_out,), lambda b: (0,)),
        ],
        out_specs=pl.BlockSpec((1, M, C_out), lambda b: (b, 0, 0)),
        out_shape=jax.ShapeDtypeStruct((B, M, C_out), jnp.float32),
    )(hmax, hmin, mu2, inv2)
    return out


def kernel(pos, feat, n_point, W1, b1, gamma1, beta1, W2, b2, gamma2, beta2):
    B, N, _ = pos.shape
    centroids = jnp.sort(_fps_jax(pos, N_POINT_STATIC), axis=1)
    center_pos = jax.vmap(lambda p, c: p[c])(pos, centroids)
    sqrd = (-2.0 * jnp.einsum('bmd,bnd->bmn', center_pos, pos)
            + jnp.sum(center_pos ** 2, -1)[:, :, None]
            + jnp.sum(pos ** 2, -1)[:, None, :])
    group_idx = jnp.argsort(jax.lax.stop_gradient(sqrd), axis=-1)[:, :, :N_NEIGHBOR]
    nbr_feat = jax.vmap(lambda f, g: f[g])(feat, group_idx)
    center_feat = jax.vmap(lambda f, c: f[c])(feat, centroids)
    new_feat = _knnconv(nbr_feat, center_feat, W1, b1, gamma1, beta1,
                        W2, b2, gamma2, beta2)
    return (center_pos, new_feat)


# speedup vs baseline: 1.6024x; 1.6003x over previous
import functools
from typing import Any

import jax
import jax.numpy as jnp
import numpy as np
from jax.experimental import pallas as pl
from jax.experimental.pallas import tpu as pltpu

N_NEIGHBOR = 64
N_POINT_STATIC = 2048


def _fps_jax(pos, n_point_static):
    pos = jax.lax.stop_gradient(pos)
    B, N, _ = pos.shape

    def body(i, state):
        centroids, dists, farthest = state
        centroids = centroids.at[:, i].set(farthest)
        cpos = jnp.take_along_axis(pos, farthest[:, None, None].astype(jnp.int32), axis=1)
        d = jnp.sum((pos - cpos) ** 2, axis=-1)
        dists = jnp.minimum(dists, d)
        farthest = jnp.argmax(dists, axis=-1).astype(jnp.int32)
        return (centroids, dists, farthest)

    centroids = jnp.zeros((B, n_point_static), dtype=jnp.int32)
    dists = jnp.full((B, N), 1e10, dtype=jnp.float32)
    farthest = jnp.zeros((B,), dtype=jnp.int32)
    centroids, _, _ = jax.lax.fori_loop(0, n_point_static, body, (centroids, dists, farthest))
    return centroids


_FS, _FL = 8, 1024


def _fps_body(planes_ref, rows_ref, cent_ref, *, B, n_iter):
    n_idx = (jax.lax.broadcasted_iota(jnp.int32, (_FS, _FL), 0) * _FL
             + jax.lax.broadcasted_iota(jnp.int32, (_FS, _FL), 1))
    c_idx = (jax.lax.broadcasted_iota(jnp.int32, (_FS, 256), 0) * 256
             + jax.lax.broadcasted_iota(jnp.int32, (_FS, 256), 1))
    planes = [planes_ref[b] for b in range(B)]

    def body(i, state):
        new_state = []
        for b in range(B):
            dists, buf, far = state[b]
            buf = jnp.where(c_idx == i, far, buf)
            cp = rows_ref[b, pl.ds(far, 1), :]
            cx, cy, cz = cp[0, 0], cp[0, 1], cp[0, 2]
            dx = planes[b][0] - cx
            dy = planes[b][1] - cy
            dz = planes[b][2] - cz
            d = dx * dx + dy * dy
            d = d + dz * dz
            dists = jnp.minimum(dists, d)
            m = jnp.max(dists)
            far = jnp.min(jnp.where(dists == m, n_idx, jnp.int32(2**30)))
            new_state.append((dists, buf, far))
        return tuple(new_state)

    init = tuple(
        (jnp.full((_FS, _FL), 1e10, dtype=jnp.float32),
         jnp.zeros((_FS, 256), dtype=jnp.int32),
         jnp.int32(0))
        for _ in range(B))
    state = jax.lax.fori_loop(0, n_iter, body, init)
    for b in range(B):
        cent_ref[b] = state[b][1]


def _fps_pallas(pos, n_point_static):
    B, N, _ = pos.shape
    planes = pos.transpose(0, 2, 1).reshape(B, 3, _FS, _FL)
    cent = pl.pallas_call(
        functools.partial(_fps_body, B=B, n_iter=n_point_static),
        in_specs=[
            pl.BlockSpec((B, 3, _FS, _FL), lambda: (0, 0, 0, 0)),
            pl.BlockSpec((B, N, 3), lambda: (0, 0, 0)),
        ],
        out_specs=pl.BlockSpec((B, _FS, 256), lambda: (0, 0, 0)),
        out_shape=jax.ShapeDtypeStruct((B, _FS, 256), jnp.int32),
    )(planes, pos)
    return cent.reshape(B, n_point_static)


_MB = 128


def _stats1_body(nbr_ref, cen_ref, wn_ref, w1a_ref, b1_ref, sum_ref, ssq_ref,
                 acc_s, acc_q):
    b = pl.program_id(0)
    m = pl.program_id(1)
    step = b * pl.num_programs(1) + m
    nbr = nbr_ref[0]
    cen = cen_ref[0]
    wn = wn_ref[...]
    w1a = w1a_ref[...]
    rows = nbr.reshape(_MB * N_NEIGHBOR, nbr.shape[-1])
    h = jnp.dot(rows, wn, preferred_element_type=jnp.float32)
    bias = b1_ref[...] - jnp.dot(cen, w1a, preferred_element_type=jnp.float32)
    h = h.reshape(_MB, N_NEIGHBOR, h.shape[-1]) + bias[:, None, :]
    s = jnp.sum(h, axis=(0, 1), keepdims=False)[None, :]
    q = jnp.sum(h * h, axis=(0, 1), keepdims=False)[None, :]

    @pl.when(step == 0)
    def _():
        acc_s[...] = jnp.zeros_like(acc_s)
        acc_q[...] = jnp.zeros_like(acc_q)

    acc_s[0:1, :] += s
    acc_q[0:1, :] += q

    @pl.when(step == pl.num_programs(0) * pl.num_programs(1) - 1)
    def _():
        sum_ref[...] = acc_s[0:1, :]
        ssq_ref[...] = acc_q[0:1, :]


def _layer2_body(nbr_ref, cen_ref, wn_ref, w1a_ref, b1_ref, mu1_ref, is1_ref,
                 w2_ref, b2_ref, hmax_ref, hmin_ref, sum_ref, ssq_ref,
                 acc_s, acc_q):
    b = pl.program_id(0)
    m = pl.program_id(1)
    step = b * pl.num_programs(1) + m
    nbr = nbr_ref[0]
    cen = cen_ref[0]
    rows = nbr.reshape(_MB * N_NEIGHBOR, nbr.shape[-1])
    h = jnp.dot(rows, wn_ref[...], preferred_element_type=jnp.float32)
    bias = b1_ref[...] - jnp.dot(cen, w1a_ref[...], preferred_element_type=jnp.float32)
    h = h.reshape(_MB, N_NEIGHBOR, h.shape[-1]) + bias[:, None, :]
    h = jnp.maximum(h * is1_ref[...][None, :] + mu1_ref[...][None, :], 0.0)
    h2 = jnp.dot(h.reshape(_MB * N_NEIGHBOR, h.shape[-1]), w2_ref[...],
                 preferred_element_type=jnp.float32) + b2_ref[...]
    s = jnp.sum(h2, axis=0)[None, :]
    q = jnp.sum(h2 * h2, axis=0)[None, :]
    h2 = h2.reshape(_MB, N_NEIGHBOR, h2.shape[-1])
    hmax_ref[0] = jnp.max(h2, axis=1)
    hmin_ref[0] = jnp.min(h2, axis=1)

    @pl.when(step == 0)
    def _():
        acc_s[...] = jnp.zeros_like(acc_s)
        acc_q[...] = jnp.zeros_like(acc_q)

    acc_s[0:1, :] += s
    acc_q[0:1, :] += q

    @pl.when(step == pl.num_programs(0) * pl.num_programs(1) - 1)
    def _():
        sum_ref[...] = acc_s[0:1, :]
        ssq_ref[...] = acc_q[0:1, :]


def _final_body(hmax_ref, hmin_ref, mu2_ref, is2_ref, out_ref):
    a = hmax_ref[...] * is2_ref[...] + mu2_ref[...]
    c = hmin_ref[...] * is2_ref[...] + mu2_ref[...]
    out_ref[...] = jnp.maximum(jnp.maximum(a, c), 0.0)


def _knnconv(nbr_feat, center_feat, W1, b1, gamma1, beta1, W2, b2, gamma2, beta2):
    B, M, K, C = nbr_feat.shape
    C_out = W1.shape[0]
    W1a = W1[:, :C]
    W1b = W1[:, C:]
    Wn = (W1a + W1b).T
    W1aT = W1a.T
    count = float(B * M * K)

    grid = (B, M // _MB)
    stats = pl.pallas_call(
        _stats1_body,
        grid=grid,
        in_specs=[
            pl.BlockSpec((1, _MB, K, C), lambda b, m: (b, m, 0, 0)),
            pl.BlockSpec((1, _MB, C), lambda b, m: (b, m, 0)),
            pl.BlockSpec((C, C_out), lambda b, m: (0, 0)),
            pl.BlockSpec((C, C_out), lambda b, m: (0, 0)),
            pl.BlockSpec((C_out,), lambda b, m: (0,)),
        ],
        out_specs=[
            pl.BlockSpec((1, C_out), lambda b, m: (0, 0)),
            pl.BlockSpec((1, C_out), lambda b, m: (0, 0)),
        ],
        out_shape=[
            jax.ShapeDtypeStruct((1, C_out), jnp.float32),
            jax.ShapeDtypeStruct((1, C_out), jnp.float32),
        ],
        scratch_shapes=[
            pltpu.VMEM((8, C_out), jnp.float32),
            pltpu.VMEM((8, C_out), jnp.float32),
        ],
    )(nbr_feat, center_feat, Wn, W1aT, b1)
    s1, q1 = stats[0][0], stats[1][0]
    mean1 = s1 / count
    var1 = q1 / count - mean1 * mean1
    inv1 = gamma1 / jnp.sqrt(var1 + 1e-5)
    mu1 = beta1 - mean1 * inv1

    hmax, hmin, s2m, q2m = pl.pallas_call(
        _layer2_body,
        grid=grid,
        in_specs=[
            pl.BlockSpec((1, _MB, K, C), lambda b, m: (b, m, 0, 0)),
            pl.BlockSpec((1, _MB, C), lambda b, m: (b, m, 0)),
            pl.BlockSpec((C, C_out), lambda b, m: (0, 0)),
            pl.BlockSpec((C, C_out), lambda b, m: (0, 0)),
            pl.BlockSpec((C_out,), lambda b, m: (0,)),
            pl.BlockSpec((C_out,), lambda b, m: (0,)),
            pl.BlockSpec((C_out,), lambda b, m: (0,)),
            pl.BlockSpec((C_out, C_out), lambda b, m: (0, 0)),
            pl.BlockSpec((C_out,), lambda b, m: (0,)),
        ],
        out_specs=[
            pl.BlockSpec((1, _MB, C_out), lambda b, m: (b, m, 0)),
            pl.BlockSpec((1, _MB, C_out), lambda b, m: (b, m, 0)),
            pl.BlockSpec((1, C_out), lambda b, m: (0, 0)),
            pl.BlockSpec((1, C_out), lambda b, m: (0, 0)),
        ],
        out_shape=[
            jax.ShapeDtypeStruct((B, M, C_out), jnp.float32),
            jax.ShapeDtypeStruct((B, M, C_out), jnp.float32),
            jax.ShapeDtypeStruct((1, C_out), jnp.float32),
            jax.ShapeDtypeStruct((1, C_out), jnp.float32),
        ],
        scratch_shapes=[
            pltpu.VMEM((8, C_out), jnp.float32),
            pltpu.VMEM((8, C_out), jnp.float32),
        ],
    )(nbr_feat, center_feat, Wn, W1aT, b1, mu1, inv1, W2.T, b2)
    s2, q2 = s2m[0], q2m[0]
    mean2 = s2 / count
    var2 = q2 / count - mean2 * mean2
    inv2 = gamma2 / jnp.sqrt(var2 + 1e-5)
    mu2 = beta2 - mean2 * inv2

    out = pl.pallas_call(
        _final_body,
        grid=(B,),
        in_specs=[
            pl.BlockSpec((1, M, C_out), lambda b: (b, 0, 0)),
            pl.BlockSpec((1, M, C_out), lambda b: (b, 0, 0)),
            pl.BlockSpec((C_out,), lambda b: (0,)),
            pl.BlockSpec((C_out,), lambda b: (0,)),
        ],
        out_specs=pl.BlockSpec((1, M, C_out), lambda b: (b, 0, 0)),
        out_shape=jax.ShapeDtypeStruct((B, M, C_out), jnp.float32),
    )(hmax, hmin, mu2, inv2)
    return out


def kernel(pos, feat, n_point, W1, b1, gamma1, beta1, W2, b2, gamma2, beta2):
    B, N, _ = pos.shape
    centroids = jnp.sort(_fps_pallas(pos, N_POINT_STATIC), axis=1)
    center_pos = jax.vmap(lambda p, c: p[c])(pos, centroids)
    sqrd = (-2.0 * jnp.einsum('bmd,bnd->bmn', center_pos, pos)
            + jnp.sum(center_pos ** 2, -1)[:, :, None]
            + jnp.sum(pos ** 2, -1)[:, None, :])
    group_idx = jnp.argsort(jax.lax.stop_gradient(sqrd), axis=-1)[:, :, :N_NEIGHBOR]
    nbr_feat = jax.vmap(lambda f, g: f[g])(feat, group_idx)
    center_feat = jax.vmap(lambda f, c: f[c])(feat, centroids)
    new_feat = _knnconv(nbr_feat, center_feat, W1, b1, gamma1, beta1,
                        W2, b2, gamma2, beta2)
    return (center_pos, new_feat)


# speedup vs baseline: 3.7989x; 2.3707x over previous
import functools
from typing import Any

import jax
import jax.numpy as jnp
import numpy as np
from jax.experimental import pallas as pl
from jax.experimental.pallas import tpu as pltpu
from jax.experimental.pallas import tpu_sc as plsc

N_NEIGHBOR = 64
N_POINT_STATIC = 2048


def _fps_jax(pos, n_point_static):
    pos = jax.lax.stop_gradient(pos)
    B, N, _ = pos.shape

    def body(i, state):
        centroids, dists, farthest = state
        centroids = centroids.at[:, i].set(farthest)
        cpos = jnp.take_along_axis(pos, farthest[:, None, None].astype(jnp.int32), axis=1)
        d = jnp.sum((pos - cpos) ** 2, axis=-1)
        dists = jnp.minimum(dists, d)
        farthest = jnp.argmax(dists, axis=-1).astype(jnp.int32)
        return (centroids, dists, farthest)

    centroids = jnp.zeros((B, n_point_static), dtype=jnp.int32)
    dists = jnp.full((B, N), 1e10, dtype=jnp.float32)
    farthest = jnp.zeros((B,), dtype=jnp.int32)
    centroids, _, _ = jax.lax.fori_loop(0, n_point_static, body, (centroids, dists, farthest))
    return centroids


_FS, _FL = 8, 1024


def _fps_body(planes_ref, rows_ref, cent_ref, *, B, n_iter):
    n_idx = (jax.lax.broadcasted_iota(jnp.int32, (_FS, _FL), 0) * _FL
             + jax.lax.broadcasted_iota(jnp.int32, (_FS, _FL), 1))
    c_idx = (jax.lax.broadcasted_iota(jnp.int32, (_FS, 256), 0) * 256
             + jax.lax.broadcasted_iota(jnp.int32, (_FS, 256), 1))
    planes = [planes_ref[b] for b in range(B)]

    def body(i, state):
        new_state = []
        for b in range(B):
            dists, buf, far = state[b]
            buf = jnp.where(c_idx == i, far, buf)
            cp = rows_ref[b, pl.ds(far, 1), :]
            cx, cy, cz = cp[0, 0], cp[0, 1], cp[0, 2]
            dx = planes[b][0] - cx
            dy = planes[b][1] - cy
            dz = planes[b][2] - cz
            d = dx * dx + dy * dy
            d = d + dz * dz
            dists = jnp.minimum(dists, d)
            m = jnp.max(dists)
            far = jnp.min(jnp.where(dists == m, n_idx, jnp.int32(2**30)))
            new_state.append((dists, buf, far))
        return tuple(new_state)

    init = tuple(
        (jnp.full((_FS, _FL), 1e10, dtype=jnp.float32),
         jnp.zeros((_FS, 256), dtype=jnp.int32),
         jnp.int32(0))
        for _ in range(B))
    state = jax.lax.fori_loop(0, n_iter, body, init)
    for b in range(B):
        cent_ref[b] = state[b][1]


def _fps_pallas(pos, n_point_static):
    B, N, _ = pos.shape
    planes = pos.transpose(0, 2, 1).reshape(B, 3, _FS, _FL)
    cent = pl.pallas_call(
        functools.partial(_fps_body, B=B, n_iter=n_point_static),
        in_specs=[
            pl.BlockSpec((B, 3, _FS, _FL), lambda: (0, 0, 0, 0)),
            pl.BlockSpec((B, N, 3), lambda: (0, 0, 0)),
        ],
        out_specs=pl.BlockSpec((B, _FS, 256), lambda: (0, 0, 0)),
        out_shape=jax.ShapeDtypeStruct((B, _FS, 256), jnp.int32),
    )(planes, pos)
    return cent.reshape(B, n_point_static)


_DMB = 128
_CHW = 64
_NCH = 128


def _bitonic_sort_sublanes(x, S):
    s_iota = jax.lax.broadcasted_iota(jnp.int32, (S, 1), 0)
    k = 2
    while k <= S:
        j = k // 2
        while j >= 1:
            x4 = x.reshape(S // (2 * j), 2, j, x.shape[-1])
            p = jnp.concatenate([x4[:, 1:2], x4[:, 0:1]], axis=1).reshape(S, x.shape[-1])
            take_min = ((s_iota & j) == 0) == ((s_iota & k) == 0)
            x = jnp.where(take_min, jnp.minimum(x, p), jnp.maximum(x, p))
            j //= 2
        k *= 2
    return x


def _dist_body(cen_ref, cent_t_ref, post_ref, pos_ref, d_ref, t_ref):
    cen = cen_ref[0]
    post = post_ref[0]
    dots = jnp.dot(cen, post, preferred_element_type=jnp.float32)
    cnorm = jnp.sum(cen * cen, axis=1, keepdims=True)
    pnorm = jnp.sum(post * post, axis=0, keepdims=True)
    d = -2.0 * dots + cnorm + pnorm
    d_ref[0] = d

    cent = cent_t_ref[0]
    posp = pos_ref[0]
    dots_t = jnp.dot(posp, cent, preferred_element_type=jnp.float32)
    pnorm_c = jnp.sum(posp * posp, axis=1, keepdims=True)
    cnorm_r = jnp.sum(cent * cent, axis=0, keepdims=True)
    dt = -2.0 * dots_t + pnorm_c + cnorm_r
    cm = jnp.min(dt.reshape(_NCH, _CHW, _DMB), axis=1)
    cm = _bitonic_sort_sublanes(cm, _NCH)
    t_ref[0] = cm[N_NEIGHBOR - 1:N_NEIGHBOR, :]


def _dist_thresh(center_pos, pos):
    B, M, _ = center_pos.shape
    N = pos.shape[1]
    cen_pad = jnp.pad(center_pos, ((0, 0), (0, 0), (0, 5)))
    cent_pad = cen_pad.transpose(0, 2, 1)
    pos_pad = jnp.pad(pos, ((0, 0), (0, 0), (0, 5)))
    post_pad = pos_pad.transpose(0, 2, 1)
    nmb = M // _DMB
    d, t = pl.pallas_call(
        _dist_body,
        grid=(B, nmb),
        in_specs=[
            pl.BlockSpec((1, _DMB, 8), lambda b, m: (b, m, 0)),
            pl.BlockSpec((1, 8, _DMB), lambda b, m: (b, 0, m)),
            pl.BlockSpec((1, 8, N), lambda b, m: (b, 0, 0)),
            pl.BlockSpec((1, N, 8), lambda b, m: (b, 0, 0)),
        ],
        out_specs=[
            pl.BlockSpec((1, _DMB, N), lambda b, m: (b, m, 0)),
            pl.BlockSpec((1, 1, _DMB), lambda b, m: (b * pl.num_programs(1) + m, 0, 0)),
        ],
        out_shape=[
            jax.ShapeDtypeStruct((B, M, N), jnp.float32),
            jax.ShapeDtypeStruct((B * nmb, 1, _DMB), jnp.float32),
        ],
    )(cen_pad, cent_pad, post_pad, pos_pad)
    return d, t.reshape(B, M)


_TK_CAP = 256
_TK_NW = 32
_SC_L = 16


def _sc_rev(xs):
    return [jax.lax.rev(x, (0,)) for x in xs[::-1]]


def _sc_bitonic_fix(ks, vs):
    ks, vs = list(ks), list(vs)
    m = len(ks)
    g = m // 2
    while g >= 1:
        for base in range(0, m, 2 * g):
            for i in range(base, base + g):
                c = ks[i] <= ks[i + g]
                nk_lo = jnp.where(c, ks[i], ks[i + g])
                nk_hi = jnp.where(c, ks[i + g], ks[i])
                nv_lo = jnp.where(c, vs[i], vs[i + g])
                nv_hi = jnp.where(c, vs[i + g], vs[i])
                ks[i], ks[i + g] = nk_lo, nk_hi
                vs[i], vs[i + g] = nv_lo, nv_hi
        g //= 2
    out = [plsc.sort_key_val(k, v) for k, v in zip(ks, vs)]
    return [o[0] for o in out], [o[1] for o in out]


def _sc_merge(a, b, keep_hi=True, fix_lo=True):
    ak, av = a
    bk, bv = b
    rk, rv = _sc_rev(bk), _sc_rev(bv)
    lok, lov, hik, hiv = [], [], [], []
    for i in range(len(ak)):
        c = ak[i] <= rk[i]
        lok.append(jnp.where(c, ak[i], rk[i]))
        lov.append(jnp.where(c, av[i], rv[i]))
        if keep_hi:
            hik.append(jnp.where(c, rk[i], ak[i]))
            hiv.append(jnp.where(c, rv[i], av[i]))
    lo = _sc_bitonic_fix(lok, lov) if fix_lo else (lok, lov)
    if not keep_hi:
        return lo
    hi = _sc_bitonic_fix(hik, hiv)
    return (lo[0] + hi[0], lo[1] + hi[1])


def _sc_select64(cd, ci):
    nrun = _TK_CAP // _SC_L
    runs = []
    for j in range(nrun):
        k = cd[pl.ds(j * _SC_L, _SC_L)]
        v = ci[pl.ds(j * _SC_L, _SC_L)]
        sk = plsc.sort_key_val(k, v)
        runs.append(([sk[0]], [sk[1]]))
    while len(runs) > 4:
        runs = [_sc_merge(runs[i], runs[i + 1], keep_hi=True)
                for i in range(0, len(runs), 2)]
    runs = [_sc_merge(runs[0], runs[1], keep_hi=False, fix_lo=True),
            _sc_merge(runs[2], runs[3], keep_hi=False, fix_lo=True)]
    lo = _sc_merge(runs[0], runs[1], keep_hi=False, fix_lo=False)
    return lo[1]


def _topk_sc(d_flat, t_flat, R, N):
    rpw = R // _TK_NW
    nch = N // _SC_L
    mesh = plsc.VectorSubcoreMesh(core_axis_name="c", subcore_axis_name="s")

    @functools.partial(
        pl.kernel,
        out_type=jax.ShapeDtypeStruct((R * N_NEIGHBOR,), jnp.int32),
        mesh=mesh,
        compiler_params=pltpu.CompilerParams(needs_layout_passes=False),
        scratch_types=[
            pltpu.VMEM((N,), jnp.float32),
            pltpu.VMEM((N,), jnp.float32),
            pltpu.VMEM((_TK_CAP,), jnp.float32),
            pltpu.VMEM((_TK_CAP,), jnp.int32),
            pltpu.VMEM((rpw * _SC_L,), jnp.float32),
            pltpu.VMEM((rpw * N_NEIGHBOR,), jnp.int32),
            pltpu.SemaphoreType.DMA,
            pltpu.SemaphoreType.DMA,
        ],
    )
    def _body(d_hbm, t_hbm, out_hbm, db0, db1, cd, ci, tb, ob, sem0, sem1):
        wid = jax.lax.axis_index("s") * 2 + jax.lax.axis_index("c")
        row0 = wid * rpw
        pltpu.sync_copy(t_hbm.at[pl.ds(row0 * _SC_L, rpw * _SC_L)], tb)
        pltpu.async_copy(d_hbm.at[pl.ds(row0 * N, N)], db0, sem0)
        pltpu.async_copy(d_hbm.at[pl.ds((row0 + 1) * N, N)], db1, sem1)
        base_iota = jax.lax.iota(jnp.int32, _SC_L)
        inf16 = jnp.full((_SC_L,), jnp.inf, dtype=jnp.float32)

        def do_row(r, dref, sem, other_r, other_dref, other_sem):
            pltpu.make_async_copy(d_hbm.at[pl.ds((row0 + r) * N, N)], dref, sem).wait()
            tvec = tb[pl.ds(r * _SC_L, _SC_L)]
            for j in range(_TK_CAP // _SC_L):
                cd[pl.ds(j * _SC_L, _SC_L)] = inf16

            def cbody(c, off):
                v = dref[pl.ds(c * _SC_L, _SC_L)]
                msk = v <= tvec
                keys = jnp.where(msk, v, jnp.inf)
                sk, si = plsc.sort_key_val(keys, base_iota + c * _SC_L)
                offc = jnp.minimum(off, _TK_CAP - _SC_L)
                cd[pl.ds(offc, _SC_L)] = sk
                ci[pl.ds(offc, _SC_L)] = si
                return off + jnp.sum(msk.astype(jnp.int32))

            jax.lax.fori_loop(0, nch, cbody, jnp.int32(0))
            @pl.when(other_r < rpw)
            def _():
                pltpu.async_copy(d_hbm.at[pl.ds((row0 + other_r) * N, N)],
                                 other_dref, other_sem)
            idx4 = _sc_select64(cd, ci)
            for q in range(4):
                ob[pl.ds(r * N_NEIGHBOR + q * _SC_L, _SC_L)] = idx4[q]

        def gbody(g, carry):
            r0 = g * 2
            do_row(r0, db0, sem0, r0 + 2, db0, sem0)
            do_row(r0 + 1, db1, sem1, r0 + 3, db1, sem1)
            return carry

        jax.lax.fori_loop(0, rpw // 2, gbody, jnp.int32(0))
        pltpu.sync_copy(ob, out_hbm.at[pl.ds(row0 * N_NEIGHBOR, rpw * N_NEIGHBOR)])

    t_rep = jnp.broadcast_to(t_flat[:, None], (R, _SC_L)).reshape(R * _SC_L)
    return _body(d_flat, t_rep)


_MB = 128


def _stats1_body(nbr_ref, cen_ref, wn_ref, w1a_ref, b1_ref, sum_ref, ssq_ref,
                 acc_s, acc_q):
    b = pl.program_id(0)
    m = pl.program_id(1)
    step = b * pl.num_programs(1) + m
    nbr = nbr_ref[0]
    cen = cen_ref[0]
    wn = wn_ref[...]
    w1a = w1a_ref[...]
    rows = nbr.reshape(_MB * N_NEIGHBOR, nbr.shape[-1])
    h = jnp.dot(rows, wn, preferred_element_type=jnp.float32)
    bias = b1_ref[...] - jnp.dot(cen, w1a, preferred_element_type=jnp.float32)
    h = h.reshape(_MB, N_NEIGHBOR, h.shape[-1]) + bias[:, None, :]
    s = jnp.sum(h, axis=(0, 1), keepdims=False)[None, :]
    q = jnp.sum(h * h, axis=(0, 1), keepdims=False)[None, :]

    @pl.when(step == 0)
    def _():
        acc_s[...] = jnp.zeros_like(acc_s)
        acc_q[...] = jnp.zeros_like(acc_q)

    acc_s[0:1, :] += s
    acc_q[0:1, :] += q

    @pl.when(step == pl.num_programs(0) * pl.num_programs(1) - 1)
    def _():
        sum_ref[...] = acc_s[0:1, :]
        ssq_ref[...] = acc_q[0:1, :]


def _layer2_body(nbr_ref, cen_ref, wn_ref, w1a_ref, b1_ref, mu1_ref, is1_ref,
                 w2_ref, b2_ref, hmax_ref, hmin_ref, sum_ref, ssq_ref,
                 acc_s, acc_q):
    b = pl.program_id(0)
    m = pl.program_id(1)
    step = b * pl.num_programs(1) + m
    nbr = nbr_ref[0]
    cen = cen_ref[0]
    rows = nbr.reshape(_MB * N_NEIGHBOR, nbr.shape[-1])
    h = jnp.dot(rows, wn_ref[...], preferred_element_type=jnp.float32)
    bias = b1_ref[...] - jnp.dot(cen, w1a_ref[...], preferred_element_type=jnp.float32)
    h = h.reshape(_MB, N_NEIGHBOR, h.shape[-1]) + bias[:, None, :]
    h = jnp.maximum(h * is1_ref[...][None, :] + mu1_ref[...][None, :], 0.0)
    h2 = jnp.dot(h.reshape(_MB * N_NEIGHBOR, h.shape[-1]), w2_ref[...],
                 preferred_element_type=jnp.float32) + b2_ref[...]
    s = jnp.sum(h2, axis=0)[None, :]
    q = jnp.sum(h2 * h2, axis=0)[None, :]
    h2 = h2.reshape(_MB, N_NEIGHBOR, h2.shape[-1])
    hmax_ref[0] = jnp.max(h2, axis=1)
    hmin_ref[0] = jnp.min(h2, axis=1)

    @pl.when(step == 0)
    def _():
        acc_s[...] = jnp.zeros_like(acc_s)
        acc_q[...] = jnp.zeros_like(acc_q)

    acc_s[0:1, :] += s
    acc_q[0:1, :] += q

    @pl.when(step == pl.num_programs(0) * pl.num_programs(1) - 1)
    def _():
        sum_ref[...] = acc_s[0:1, :]
        ssq_ref[...] = acc_q[0:1, :]


def _final_body(hmax_ref, hmin_ref, mu2_ref, is2_ref, out_ref):
    a = hmax_ref[...] * is2_ref[...] + mu2_ref[...]
    c = hmin_ref[...] * is2_ref[...] + mu2_ref[...]
    out_ref[...] = jnp.maximum(jnp.maximum(a, c), 0.0)


def _knnconv(nbr_feat, center_feat, W1, b1, gamma1, beta1, W2, b2, gamma2, beta2):
    B, M, K, C = nbr_feat.shape
    C_out = W1.shape[0]
    W1a = W1[:, :C]
    W1b = W1[:, C:]
    Wn = (W1a + W1b).T
    W1aT = W1a.T
    count = float(B * M * K)

    grid = (B, M // _MB)
    stats = pl.pallas_call(
        _stats1_body,
        grid=grid,
        in_specs=[
            pl.BlockSpec((1, _MB, K, C), lambda b, m: (b, m, 0, 0)),
            pl.BlockSpec((1, _MB, C), lambda b, m: (b, m, 0)),
            pl.BlockSpec((C, C_out), lambda b, m: (0, 0)),
            pl.BlockSpec((C, C_out), lambda b, m: (0, 0)),
            pl.BlockSpec((C_out,), lambda b, m: (0,)),
        ],
        out_specs=[
            pl.BlockSpec((1, C_out), lambda b, m: (0, 0)),
            pl.BlockSpec((1, C_out), lambda b, m: (0, 0)),
        ],
        out_shape=[
            jax.ShapeDtypeStruct((1, C_out), jnp.float32),
            jax.ShapeDtypeStruct((1, C_out), jnp.float32),
        ],
        scratch_shapes=[
            pltpu.VMEM((8, C_out), jnp.float32),
            pltpu.VMEM((8, C_out), jnp.float32),
        ],
    )(nbr_feat, center_feat, Wn, W1aT, b1)
    s1, q1 = stats[0][0], stats[1][0]
    mean1 = s1 / count
    var1 = q1 / count - mean1 * mean1
    inv1 = gamma1 / jnp.sqrt(var1 + 1e-5)
    mu1 = beta1 - mean1 * inv1

    hmax, hmin, s2m, q2m = pl.pallas_call(
        _layer2_body,
        grid=grid,
        in_specs=[
            pl.BlockSpec((1, _MB, K, C), lambda b, m: (b, m, 0, 0)),
            pl.BlockSpec((1, _MB, C), lambda b, m: (b, m, 0)),
            pl.BlockSpec((C, C_out), lambda b, m: (0, 0)),
            pl.BlockSpec((C, C_out), lambda b, m: (0, 0)),
            pl.BlockSpec((C_out,), lambda b, m: (0,)),
            pl.BlockSpec((C_out,), lambda b, m: (0,)),
            pl.BlockSpec((C_out,), lambda b, m: (0,)),
            pl.BlockSpec((C_out, C_out), lambda b, m: (0, 0)),
            pl.BlockSpec((C_out,), lambda b, m: (0,)),
        ],
        out_specs=[
            pl.BlockSpec((1, _MB, C_out), lambda b, m: (b, m, 0)),
            pl.BlockSpec((1, _MB, C_out), lambda b, m: (b, m, 0)),
            pl.BlockSpec((1, C_out), lambda b, m: (0, 0)),
            pl.BlockSpec((1, C_out), lambda b, m: (0, 0)),
        ],
        out_shape=[
            jax.ShapeDtypeStruct((B, M, C_out), jnp.float32),
            jax.ShapeDtypeStruct((B, M, C_out), jnp.float32),
            jax.ShapeDtypeStruct((1, C_out), jnp.float32),
            jax.ShapeDtypeStruct((1, C_out), jnp.float32),
        ],
        scratch_shapes=[
            pltpu.VMEM((8, C_out), jnp.float32),
            pltpu.VMEM((8, C_out), jnp.float32),
        ],
    )(nbr_feat, center_feat, Wn, W1aT, b1, mu1, inv1, W2.T, b2)
    s2, q2 = s2m[0], q2m[0]
    mean2 = s2 / count
    var2 = q2 / count - mean2 * mean2
    inv2 = gamma2 / jnp.sqrt(var2 + 1e-5)
    mu2 = beta2 - mean2 * inv2

    out = pl.pallas_call(
        _final_body,
        grid=(B,),
        in_specs=[
            pl.BlockSpec((1, M, C_out), lambda b: (b, 0, 0)),
            pl.BlockSpec((1, M, C_out), lambda b: (b, 0, 0)),
            pl.BlockSpec((C_out,), lambda b: (0,)),
            pl.BlockSpec((C_out,), lambda b: (0,)),
        ],
        out_specs=pl.BlockSpec((1, M, C_out), lambda b: (b, 0, 0)),
        out_shape=jax.ShapeDtypeStruct((B, M, C_out), jnp.float32),
    )(hmax, hmin, mu2, inv2)
    return out


def kernel(pos, feat, n_point, W1, b1, gamma1, beta1, W2, b2, gamma2, beta2):
    B, N, _ = pos.shape
    centroids = jnp.sort(_fps_pallas(pos, N_POINT_STATIC), axis=1)
    center_pos = jax.vmap(lambda p, c: p[c])(pos, centroids)
    M = centroids.shape[1]
    D, T = _dist_thresh(center_pos, pos)
    group_idx = _topk_sc(D.reshape(B * M * N), T.reshape(B * M),
                         B * M, N).reshape(B, M, N_NEIGHBOR)
    nbr_feat = jax.vmap(lambda f, g: f[g])(feat, group_idx)
    center_feat = jax.vmap(lambda f, c: f[c])(feat, centroids)
    new_feat = _knnconv(nbr_feat, center_feat, W1, b1, gamma1, beta1,
                        W2, b2, gamma2, beta2)
    return (center_pos, new_feat)


# speedup vs baseline: 9.6157x; 2.5312x over previous
import functools
from typing import Any

import jax
import jax.numpy as jnp
import numpy as np
from jax.experimental import pallas as pl
from jax.experimental.pallas import tpu as pltpu
from jax.experimental.pallas import tpu_sc as plsc

N_NEIGHBOR = 64
N_POINT_STATIC = 2048


def _fps_jax(pos, n_point_static):
    pos = jax.lax.stop_gradient(pos)
    B, N, _ = pos.shape

    def body(i, state):
        centroids, dists, farthest = state
        centroids = centroids.at[:, i].set(farthest)
        cpos = jnp.take_along_axis(pos, farthest[:, None, None].astype(jnp.int32), axis=1)
        d = jnp.sum((pos - cpos) ** 2, axis=-1)
        dists = jnp.minimum(dists, d)
        farthest = jnp.argmax(dists, axis=-1).astype(jnp.int32)
        return (centroids, dists, farthest)

    centroids = jnp.zeros((B, n_point_static), dtype=jnp.int32)
    dists = jnp.full((B, N), 1e10, dtype=jnp.float32)
    farthest = jnp.zeros((B,), dtype=jnp.int32)
    centroids, _, _ = jax.lax.fori_loop(0, n_point_static, body, (centroids, dists, farthest))
    return centroids


_FS, _FL = 8, 1024


def _fps_body(planes_ref, rows_ref, cent_ref, *, B, n_iter):
    n_idx = (jax.lax.broadcasted_iota(jnp.int32, (_FS, _FL), 0) * _FL
             + jax.lax.broadcasted_iota(jnp.int32, (_FS, _FL), 1))
    c_idx = (jax.lax.broadcasted_iota(jnp.int32, (_FS, 256), 0) * 256
             + jax.lax.broadcasted_iota(jnp.int32, (_FS, 256), 1))
    planes = [planes_ref[b] for b in range(B)]

    def body(i, state):
        new_state = []
        for b in range(B):
            dists, buf, far = state[b]
            buf = jnp.where(c_idx == i, far, buf)
            cp = rows_ref[b, pl.ds(far, 1), :]
            cx, cy, cz = cp[0, 0], cp[0, 1], cp[0, 2]
            dx = planes[b][0] - cx
            dy = planes[b][1] - cy
            dz = planes[b][2] - cz
            d = dx * dx + dy * dy
            d = d + dz * dz
            dists = jnp.minimum(dists, d)
            m = jnp.max(dists)
            far = jnp.min(jnp.where(dists == m, n_idx, jnp.int32(2**30)))
            new_state.append((dists, buf, far))
        return tuple(new_state)

    init = tuple(
        (jnp.full((_FS, _FL), 1e10, dtype=jnp.float32),
         jnp.zeros((_FS, 256), dtype=jnp.int32),
         jnp.int32(0))
        for _ in range(B))
    state = jax.lax.fori_loop(0, n_iter, body, init)
    for b in range(B):
        cent_ref[b] = state[b][1]


def _fps_pallas(pos, n_point_static):
    B, N, _ = pos.shape
    planes = pos.transpose(0, 2, 1).reshape(B, 3, _FS, _FL)
    cent = pl.pallas_call(
        functools.partial(_fps_body, B=B, n_iter=n_point_static),
        in_specs=[
            pl.BlockSpec((B, 3, _FS, _FL), lambda: (0, 0, 0, 0)),
            pl.BlockSpec((B, N, 3), lambda: (0, 0, 0)),
        ],
        out_specs=pl.BlockSpec((B, _FS, 256), lambda: (0, 0, 0)),
        out_shape=jax.ShapeDtypeStruct((B, _FS, 256), jnp.int32),
    )(planes, pos)
    return cent.reshape(B, n_point_static)


_DMB = 128
_CHW = 64
_NCH = 128


def _bitonic_sort_sublanes(x, S):
    s_iota = jax.lax.broadcasted_iota(jnp.int32, (S, 1), 0)
    k = 2
    while k <= S:
        j = k // 2
        while j >= 1:
            x4 = x.reshape(S // (2 * j), 2, j, x.shape[-1])
            p = jnp.concatenate([x4[:, 1:2], x4[:, 0:1]], axis=1).reshape(S, x.shape[-1])
            take_min = ((s_iota & j) == 0) == ((s_iota & k) == 0)
            x = jnp.where(take_min, jnp.minimum(x, p), jnp.maximum(x, p))
            j //= 2
        k *= 2
    return x


def _dist_body(cen_ref, cent_t_ref, post_ref, pos_ref, d_ref, t_ref):
    cen = cen_ref[0]
    post = post_ref[0]
    dots = jnp.dot(cen, post, preferred_element_type=jnp.float32)
    cnorm = jnp.sum(cen * cen, axis=1, keepdims=True)
    pnorm = jnp.sum(post * post, axis=0, keepdims=True)
    d = -2.0 * dots + cnorm + pnorm
    d_ref[0] = d

    cent = cent_t_ref[0]
    posp = pos_ref[0]
    dots_t = jnp.dot(posp, cent, preferred_element_type=jnp.float32)
    pnorm_c = jnp.sum(posp * posp, axis=1, keepdims=True)
    cnorm_r = jnp.sum(cent * cent, axis=0, keepdims=True)
    dt = -2.0 * dots_t + pnorm_c + cnorm_r
    cm = jnp.min(dt.reshape(_NCH, _CHW, _DMB), axis=1)
    cm = _bitonic_sort_sublanes(cm, _NCH)
    t_ref[0] = cm[N_NEIGHBOR - 1:N_NEIGHBOR, :]


def _dist_thresh(center_pos, pos):
    B, M, _ = center_pos.shape
    N = pos.shape[1]
    cen_pad = jnp.pad(center_pos, ((0, 0), (0, 0), (0, 5)))
    cent_pad = cen_pad.transpose(0, 2, 1)
    pos_pad = jnp.pad(pos, ((0, 0), (0, 0), (0, 5)))
    post_pad = pos_pad.transpose(0, 2, 1)
    nmb = M // _DMB
    d, t = pl.pallas_call(
        _dist_body,
        grid=(B, nmb),
        in_specs=[
            pl.BlockSpec((1, _DMB, 8), lambda b, m: (b, m, 0)),
            pl.BlockSpec((1, 8, _DMB), lambda b, m: (b, 0, m)),
            pl.BlockSpec((1, 8, N), lambda b, m: (b, 0, 0)),
            pl.BlockSpec((1, N, 8), lambda b, m: (b, 0, 0)),
        ],
        out_specs=[
            pl.BlockSpec((1, _DMB, N), lambda b, m: (b, m, 0)),
            pl.BlockSpec((1, 1, _DMB), lambda b, m: (b * pl.num_programs(1) + m, 0, 0)),
        ],
        out_shape=[
            jax.ShapeDtypeStruct((B, M, N), jnp.float32),
            jax.ShapeDtypeStruct((B * nmb, 1, _DMB), jnp.float32),
        ],
    )(cen_pad, cent_pad, post_pad, pos_pad)
    return d, t.reshape(B, M)


_TK_CAP = 256
_TK_NW = 32
_SC_L = 16


def _sc_rev(xs):
    return [jax.lax.rev(x, (0,)) for x in xs[::-1]]


def _sc_bitonic_fix(ks, vs):
    ks, vs = list(ks), list(vs)
    m = len(ks)
    g = m // 2
    while g >= 1:
        for base in range(0, m, 2 * g):
            for i in range(base, base + g):
                c = ks[i] <= ks[i + g]
                nk_lo = jnp.where(c, ks[i], ks[i + g])
                nk_hi = jnp.where(c, ks[i + g], ks[i])
                nv_lo = jnp.where(c, vs[i], vs[i + g])
                nv_hi = jnp.where(c, vs[i + g], vs[i])
                ks[i], ks[i + g] = nk_lo, nk_hi
                vs[i], vs[i + g] = nv_lo, nv_hi
        g //= 2
    out = [plsc.sort_key_val(k, v) for k, v in zip(ks, vs)]
    return [o[0] for o in out], [o[1] for o in out]


def _sc_merge(a, b, keep_hi=True, fix_lo=True):
    ak, av = a
    bk, bv = b
    rk, rv = _sc_rev(bk), _sc_rev(bv)
    lok, lov, hik, hiv = [], [], [], []
    for i in range(len(ak)):
        c = ak[i] <= rk[i]
        lok.append(jnp.where(c, ak[i], rk[i]))
        lov.append(jnp.where(c, av[i], rv[i]))
        if keep_hi:
            hik.append(jnp.where(c, rk[i], ak[i]))
            hiv.append(jnp.where(c, rv[i], av[i]))
    lo = _sc_bitonic_fix(lok, lov) if fix_lo else (lok, lov)
    if not keep_hi:
        return lo
    hi = _sc_bitonic_fix(hik, hiv)
    return (lo[0] + hi[0], lo[1] + hi[1])


def _sc_select64(cd, ci):
    nrun = _TK_CAP // _SC_L
    runs = []
    for j in range(nrun):
        k = cd[pl.ds(j * _SC_L, _SC_L)]
        v = ci[pl.ds(j * _SC_L, _SC_L)]
        sk = plsc.sort_key_val(k, v)
        runs.append(([sk[0]], [sk[1]]))
    while len(runs) > 4:
        runs = [_sc_merge(runs[i], runs[i + 1], keep_hi=True)
                for i in range(0, len(runs), 2)]
    runs = [_sc_merge(runs[0], runs[1], keep_hi=False, fix_lo=True),
            _sc_merge(runs[2], runs[3], keep_hi=False, fix_lo=True)]
    lo = _sc_merge(runs[0], runs[1], keep_hi=False, fix_lo=False)
    return lo[1]


def _topk_sc(d_flat, t_flat, R, N):
    rpw = R // _TK_NW
    nch = N // _SC_L
    mesh = plsc.VectorSubcoreMesh(core_axis_name="c", subcore_axis_name="s")

    @functools.partial(
        pl.kernel,
        out_type=jax.ShapeDtypeStruct((R * N_NEIGHBOR,), jnp.int32),
        mesh=mesh,
        compiler_params=pltpu.CompilerParams(needs_layout_passes=False),
        scratch_types=[
            pltpu.VMEM((N,), jnp.float32),
            pltpu.VMEM((N,), jnp.float32),
            pltpu.VMEM((_TK_CAP,), jnp.float32),
            pltpu.VMEM((_TK_CAP,), jnp.int32),
            pltpu.VMEM((rpw * _SC_L,), jnp.float32),
            pltpu.VMEM((rpw * N_NEIGHBOR,), jnp.int32),
            pltpu.SemaphoreType.DMA,
            pltpu.SemaphoreType.DMA,
        ],
    )
    def _body(d_hbm, t_hbm, out_hbm, db0, db1, cd, ci, tb, ob, sem0, sem1):
        wid = jax.lax.axis_index("s") * 2 + jax.lax.axis_index("c")
        row0 = wid * rpw
        gbase = (row0 // 2048) * N
        pltpu.sync_copy(t_hbm.at[pl.ds(row0 * _SC_L, rpw * _SC_L)], tb)
        pltpu.async_copy(d_hbm.at[pl.ds(row0 * N, N)], db0, sem0)
        pltpu.async_copy(d_hbm.at[pl.ds((row0 + 1) * N, N)], db1, sem1)
        base_iota = jax.lax.iota(jnp.int32, _SC_L)
        inf16 = jnp.full((_SC_L,), jnp.inf, dtype=jnp.float32)

        def do_row(r, dref, sem, other_r, other_dref, other_sem):
            pltpu.make_async_copy(d_hbm.at[pl.ds((row0 + r) * N, N)], dref, sem).wait()
            tvec = tb[pl.ds(r * _SC_L, _SC_L)]
            for j in range(_TK_CAP // _SC_L):
                cd[pl.ds(j * _SC_L, _SC_L)] = inf16

            def cbody(c, off):
                v = dref[pl.ds(c * _SC_L, _SC_L)]
                msk = v <= tvec
                keys = jnp.where(msk, v, jnp.inf)
                sk, si = plsc.sort_key_val(keys, base_iota + c * _SC_L)
                offc = jnp.minimum(off, _TK_CAP - _SC_L)
                cd[pl.ds(offc, _SC_L)] = sk
                ci[pl.ds(offc, _SC_L)] = si
                return off + jnp.sum(msk.astype(jnp.int32))

            jax.lax.fori_loop(0, nch, cbody, jnp.int32(0))
            @pl.when(other_r < rpw)
            def _():
                pltpu.async_copy(d_hbm.at[pl.ds((row0 + other_r) * N, N)],
                                 other_dref, other_sem)
            idx4 = _sc_select64(cd, ci)
            for q in range(4):
                ob[pl.ds(r * N_NEIGHBOR + q * _SC_L, _SC_L)] = idx4[q] + gbase

        def gbody(g, carry):
            r0 = g * 2
            do_row(r0, db0, sem0, r0 + 2, db0, sem0)
            do_row(r0 + 1, db1, sem1, r0 + 3, db1, sem1)
            return carry

        jax.lax.fori_loop(0, rpw // 2, gbody, jnp.int32(0))
        pltpu.sync_copy(ob, out_hbm.at[pl.ds(row0 * N_NEIGHBOR, rpw * N_NEIGHBOR)])

    t_rep = jnp.broadcast_to(t_flat[:, None], (R, _SC_L)).reshape(R * _SC_L)
    return _body(d_flat, t_rep)


_GCH = 1024


def _gather_sc(table, idx):
    V, C = table.shape
    NI = idx.shape[0]
    ipw = NI // _TK_NW
    nch = ipw // _GCH
    mesh = plsc.VectorSubcoreMesh(core_axis_name="c", subcore_axis_name="s")

    @functools.partial(
        pl.kernel,
        out_type=jax.ShapeDtypeStruct((NI, C), jnp.float32),
        mesh=mesh,
        compiler_params=pltpu.CompilerParams(
            needs_layout_passes=False, use_tc_tiling_on_sc=False),
        scratch_types=[
            pltpu.VMEM((ipw,), jnp.int32),
            pltpu.VMEM((_GCH, 32), jnp.float32),
            pltpu.VMEM((_GCH, 32), jnp.float32),
            pltpu.SemaphoreType.DMA,
            pltpu.SemaphoreType.DMA,
        ],
    )
    def _body(tab_hbm, idx_hbm, out_hbm, idxv, rb0, rb1, sem0, sem1):
        wid = jax.lax.axis_index("s") * 2 + jax.lax.axis_index("c")
        base = wid * ipw
        pltpu.sync_copy(idx_hbm.at[pl.ds(base, ipw)], idxv)
        pltpu.async_copy(tab_hbm.at[idxv.at[pl.ds(0, _GCH)]], rb0, sem0)
        pltpu.async_copy(tab_hbm.at[idxv.at[pl.ds(_GCH, _GCH)]], rb1, sem1)

        def do_chunk(j, rb, sem, nxt):
            pltpu.make_async_copy(
                tab_hbm.at[idxv.at[pl.ds(j * _GCH, _GCH)]], rb, sem).wait()
            pltpu.sync_copy(rb, out_hbm.at[pl.ds(base + j * _GCH, _GCH), :])
            @pl.when(nxt < nch)
            def _():
                pltpu.async_copy(
                    tab_hbm.at[idxv.at[pl.ds(nxt * _GCH, _GCH)]], rb, sem)

        def gbody(g, carry):
            do_chunk(g * 2, rb0, sem0, g * 2 + 2)
            do_chunk(g * 2 + 1, rb1, sem1, g * 2 + 3)
            return carry

        jax.lax.fori_loop(0, nch // 2, gbody, jnp.int32(0))

    return _body(table, idx)


_MB = 128


def _stats1_body(nbr_ref, cen_ref, wn_ref, w1a_ref, b1_ref, sum_ref, ssq_ref,
                 acc_s, acc_q):
    b = pl.program_id(0)
    m = pl.program_id(1)
    step = b * pl.num_programs(1) + m
    nbr = nbr_ref[0]
    cen = cen_ref[0]
    wn = wn_ref[...]
    w1a = w1a_ref[...]
    rows = nbr.reshape(_MB * N_NEIGHBOR, nbr.shape[-1])
    h = jnp.dot(rows, wn, preferred_element_type=jnp.float32)
    bias = b1_ref[...] - jnp.dot(cen, w1a, preferred_element_type=jnp.float32)
    h = h.reshape(_MB, N_NEIGHBOR, h.shape[-1]) + bias[:, None, :]
    s = jnp.sum(h, axis=(0, 1), keepdims=False)[None, :]
    q = jnp.sum(h * h, axis=(0, 1), keepdims=False)[None, :]

    @pl.when(step == 0)
    def _():
        acc_s[...] = jnp.zeros_like(acc_s)
        acc_q[...] = jnp.zeros_like(acc_q)

    acc_s[0:1, :] += s
    acc_q[0:1, :] += q

    @pl.when(step == pl.num_programs(0) * pl.num_programs(1) - 1)
    def _():
        sum_ref[...] = acc_s[0:1, :]
        ssq_ref[...] = acc_q[0:1, :]


def _layer2_body(nbr_ref, cen_ref, wn_ref, w1a_ref, b1_ref, mu1_ref, is1_ref,
                 w2_ref, b2_ref, hmax_ref, hmin_ref, sum_ref, ssq_ref,
                 acc_s, acc_q):
    b = pl.program_id(0)
    m = pl.program_id(1)
    step = b * pl.num_programs(1) + m
    nbr = nbr_ref[0]
    cen = cen_ref[0]
    rows = nbr.reshape(_MB * N_NEIGHBOR, nbr.shape[-1])
    h = jnp.dot(rows, wn_ref[...], preferred_element_type=jnp.float32)
    bias = b1_ref[...] - jnp.dot(cen, w1a_ref[...], preferred_element_type=jnp.float32)
    h = h.reshape(_MB, N_NEIGHBOR, h.shape[-1]) + bias[:, None, :]
    h = jnp.maximum(h * is1_ref[...][None, :] + mu1_ref[...][None, :], 0.0)
    h2 = jnp.dot(h.reshape(_MB * N_NEIGHBOR, h.shape[-1]), w2_ref[...],
                 preferred_element_type=jnp.float32) + b2_ref[...]
    s = jnp.sum(h2, axis=0)[None, :]
    q = jnp.sum(h2 * h2, axis=0)[None, :]
    h2 = h2.reshape(_MB, N_NEIGHBOR, h2.shape[-1])
    hmax_ref[0] = jnp.max(h2, axis=1)
    hmin_ref[0] = jnp.min(h2, axis=1)

    @pl.when(step == 0)
    def _():
        acc_s[...] = jnp.zeros_like(acc_s)
        acc_q[...] = jnp.zeros_like(acc_q)

    acc_s[0:1, :] += s
    acc_q[0:1, :] += q

    @pl.when(step == pl.num_programs(0) * pl.num_programs(1) - 1)
    def _():
        sum_ref[...] = acc_s[0:1, :]
        ssq_ref[...] = acc_q[0:1, :]


def _final_body(hmax_ref, hmin_ref, mu2_ref, is2_ref, out_ref):
    a = hmax_ref[...] * is2_ref[...] + mu2_ref[...]
    c = hmin_ref[...] * is2_ref[...] + mu2_ref[...]
    out_ref[...] = jnp.maximum(jnp.maximum(a, c), 0.0)


def _knnconv(nbr_feat, center_feat, W1, b1, gamma1, beta1, W2, b2, gamma2, beta2):
    B, M, K, C = nbr_feat.shape
    C_out = W1.shape[0]
    W1a = W1[:, :C]
    W1b = W1[:, C:]
    Wn = (W1a + W1b).T
    W1aT = W1a.T
    count = float(B * M * K)

    grid = (B, M // _MB)
    stats = pl.pallas_call(
        _stats1_body,
        grid=grid,
        in_specs=[
            pl.BlockSpec((1, _MB, K, C), lambda b, m: (b, m, 0, 0)),
            pl.BlockSpec((1, _MB, C), lambda b, m: (b, m, 0)),
            pl.BlockSpec((C, C_out), lambda b, m: (0, 0)),
            pl.BlockSpec((C, C_out), lambda b, m: (0, 0)),
            pl.BlockSpec((C_out,), lambda b, m: (0,)),
        ],
        out_specs=[
            pl.BlockSpec((1, C_out), lambda b, m: (0, 0)),
            pl.BlockSpec((1, C_out), lambda b, m: (0, 0)),
        ],
        out_shape=[
            jax.ShapeDtypeStruct((1, C_out), jnp.float32),
            jax.ShapeDtypeStruct((1, C_out), jnp.float32),
        ],
        scratch_shapes=[
            pltpu.VMEM((8, C_out), jnp.float32),
            pltpu.VMEM((8, C_out), jnp.float32),
        ],
    )(nbr_feat, center_feat, Wn, W1aT, b1)
    s1, q1 = stats[0][0], stats[1][0]
    mean1 = s1 / count
    var1 = q1 / count - mean1 * mean1
    inv1 = gamma1 / jnp.sqrt(var1 + 1e-5)
    mu1 = beta1 - mean1 * inv1

    hmax, hmin, s2m, q2m = pl.pallas_call(
        _layer2_body,
        grid=grid,
        in_specs=[
            pl.BlockSpec((1, _MB, K, C), lambda b, m: (b, m, 0, 0)),
            pl.BlockSpec((1, _MB, C), lambda b, m: (b, m, 0)),
            pl.BlockSpec((C, C_out), lambda b, m: (0, 0)),
            pl.BlockSpec((C, C_out), lambda b, m: (0, 0)),
            pl.BlockSpec((C_out,), lambda b, m: (0,)),
            pl.BlockSpec((C_out,), lambda b, m: (0,)),
            pl.BlockSpec((C_out,), lambda b, m: (0,)),
            pl.BlockSpec((C_out, C_out), lambda b, m: (0, 0)),
            pl.BlockSpec((C_out,), lambda b, m: (0,)),
        ],
        out_specs=[
            pl.BlockSpec((1, _MB, C_out), lambda b, m: (b, m, 0)),
            pl.BlockSpec((1, _MB, C_out), lambda b, m: (b, m, 0)),
            pl.BlockSpec((1, C_out), lambda b, m: (0, 0)),
            pl.BlockSpec((1, C_out), lambda b, m: (0, 0)),
        ],
        out_shape=[
            jax.ShapeDtypeStruct((B, M, C_out), jnp.float32),
            jax.ShapeDtypeStruct((B, M, C_out), jnp.float32),
            jax.ShapeDtypeStruct((1, C_out), jnp.float32),
            jax.ShapeDtypeStruct((1, C_out), jnp.float32),
        ],
        scratch_shapes=[
            pltpu.VMEM((8, C_out), jnp.float32),
            pltpu.VMEM((8, C_out), jnp.float32),
        ],
    )(nbr_feat, center_feat, Wn, W1aT, b1, mu1, inv1, W2.T, b2)
    s2, q2 = s2m[0], q2m[0]
    mean2 = s2 / count
    var2 = q2 / count - mean2 * mean2
    inv2 = gamma2 / jnp.sqrt(var2 + 1e-5)
    mu2 = beta2 - mean2 * inv2

    out = pl.pallas_call(
        _final_body,
        grid=(B,),
        in_specs=[
            pl.BlockSpec((1, M, C_out), lambda b: (b, 0, 0)),
            pl.BlockSpec((1, M, C_out), lambda b: (b, 0, 0)),
            pl.BlockSpec((C_out,), lambda b: (0,)),
            pl.BlockSpec((C_out,), lambda b: (0,)),
        ],
        out_specs=pl.BlockSpec((1, M, C_out), lambda b: (b, 0, 0)),
        out_shape=jax.ShapeDtypeStruct((B, M, C_out), jnp.float32),
    )(hmax, hmin, mu2, inv2)
    return out


def kernel(pos, feat, n_point, W1, b1, gamma1, beta1, W2, b2, gamma2, beta2):
    B, N, _ = pos.shape
    centroids = jnp.sort(_fps_pallas(pos, N_POINT_STATIC), axis=1)
    center_pos = jax.vmap(lambda p, c: p[c])(pos, centroids)
    M = centroids.shape[1]
    D, T = _dist_thresh(center_pos, pos)
    group_idx = _topk_sc(D.reshape(B * M * N), T.reshape(B * M),
                         B * M, N).reshape(B, M, N_NEIGHBOR)
    nbr_feat = _gather_sc(feat.reshape(B * N, feat.shape[-1]), group_idx.reshape(B * M * N_NEIGHBOR)).reshape(B, M, N_NEIGHBOR, feat.shape[-1])
    center_feat = jax.vmap(lambda f, c: f[c])(feat, centroids)
    new_feat = _knnconv(nbr_feat, center_feat, W1, b1, gamma1, beta1,
                        W2, b2, gamma2, beta2)
    return (center_pos, new_feat)
